# baseline (speedup 1.0000x reference)
import functools

import jax
import jax.numpy as jnp
import numpy as np
from jax import lax
from jax.experimental import pallas as pl
from jax.experimental.pallas import tpu as pltpu
from jax.experimental.pallas import tpu_sc as plsc

_FEAT_STRIDE = 16
_WH = np.array([16., 32., 64., 128., 256., 512., 1024.], dtype=np.float32)
_ANCH = np.stack([-(_WH - 1.0) / 2.0, (_WH - 1.0) / 2.0], axis=1).astype(np.float32)
_PRE_TOPN = 6000
_POST_TOPN = 300
_THRESH = 0.7
_COM_TOPN = 2000
_COM_TOPK = 3

_N = 6000
_C = 256
_NC = 24
_NP = _C * _NC

_INTERPRET = False


def _decode(strings, deltas):
    widths = strings[..., 1] - strings[..., 0] + 1.0
    ctr = strings[..., 0] + 0.5 * widths
    d_ctr = deltas[..., 0]
    d_w = jnp.clip(deltas[..., 1], -10.0, 4.0)
    pred_ctr = d_ctr * widths + ctr
    pred_w = jnp.exp(d_w) * widths
    return jnp.stack([pred_ctr - 0.5 * (pred_w - 1.0),
                      pred_ctr + 0.5 * (pred_w - 1.0)], axis=-1)


def _pairs(prop_a, prop_b, sc_a, sc_b, A, K, primary_is_w):
    B = prop_a.shape[0]
    top_sa, top_ia = jax.lax.top_k(sc_a, _COM_TOPN)
    pos = top_ia // A
    a_sel = jnp.take_along_axis(prop_a, top_ia[..., None], axis=1)
    sc_b_r = sc_b.reshape(B, K, A)
    prop_b_r = prop_b.reshape(B, K, A, 2)
    top_sb, top_ib = jax.lax.top_k(sc_b_r, _COM_TOPK)
    b_strings = jnp.take_along_axis(prop_b_r, top_ib[..., None], axis=2)
    idx_s = jnp.broadcast_to(pos[:, :, None], (B, _COM_TOPN, _COM_TOPK))
    sb_at = jnp.take_along_axis(top_sb, idx_s, axis=1)
    idx_b = jnp.broadcast_to(pos[:, :, None, None], (B, _COM_TOPN, _COM_TOPK, 2))
    b_at = jnp.take_along_axis(b_strings, idx_b, axis=1)
    a_exp = jnp.broadcast_to(a_sel[:, :, None, :], (B, _COM_TOPN, _COM_TOPK, 2))
    if primary_is_w:
        boxes = jnp.stack([a_exp[..., 0], b_at[..., 0], a_exp[..., 1], b_at[..., 1]], axis=-1)
    else:
        boxes = jnp.stack([b_at[..., 0], a_exp[..., 0], b_at[..., 1], a_exp[..., 1]], axis=-1)
    scores = top_sa[:, :, None] * sb_at
    return boxes.reshape(B, _COM_TOPN * _COM_TOPK, 4), scores.reshape(B, _COM_TOPN * _COM_TOPK)


def _pairs_sc_kernel(props, scs, ias, sas, clipb, bx_out, sc_out,
                     a_buf, b_buf, scb_buf, ia_buf, sa_buf,
                     stage_bx, stage_sc, clip_buf):
    cid = lax.axis_index("c")
    sid = lax.axis_index("s")
    wid = sid * 2 + cid
    img = wid // 8
    side = (wid // 4) % 2
    part = wid % 4

    pltpu.sync_copy(props.at[img, side], a_buf)
    pltpu.sync_copy(props.at[img, 1 - side], b_buf)
    pltpu.sync_copy(scs.at[img, 1 - side], scb_buf)
    pltpu.sync_copy(ias.at[img, side, pl.ds(part * 512, 512)], ia_buf)
    pltpu.sync_copy(sas.at[img, side, pl.ds(part * 512, 512)], sa_buf)
    pltpu.sync_copy(clipb.at[img], clip_buf)

    clipx = clip_buf[0, :]
    clipy = clip_buf[1, :]
    sidew = lax.broadcast(side, (16,)) == 0
    lanes = lax.broadcasted_iota(jnp.int32, (16,), 0)
    zero16 = jnp.zeros((16,), jnp.float32)

    def body(g, carry):
        ia_v = ia_buf[pl.ds(g * 16, 16)]
        sa_v = sa_buf[pl.ds(g * 16, 16)]
        a0 = plsc.load_gather(a_buf, [ia_v * 2])
        a1 = plsc.load_gather(a_buf, [ia_v * 2 + 1])
        pos7 = (ia_v // 7) * 7
        vals = [plsc.load_gather(scb_buf, [pos7 + a]) for a in range(7)]

        picked = []
        for _ in range(3):
            m = jnp.full((16,), -1.0, jnp.float32)
            mi = jnp.zeros((16,), jnp.int32)
            for a in range(7):
                va = vals[a]
                for (_, pix) in picked:
                    va = jnp.where(pix == a, -1.0, va)
                cond = va > m
                m = jnp.where(cond, va, m)
                mi = jnp.where(cond, a, mi)
            picked.append((m, mi))

        for k, (mk, ik) in enumerate(picked):
            bidx = pos7 + ik
            b0 = plsc.load_gather(b_buf, [bidx * 2])
            b1 = plsc.load_gather(b_buf, [bidx * 2 + 1])
            sc_v = sa_v * mk
            x1 = jnp.where(sidew, a0, b0)
            y1 = jnp.where(sidew, b0, a0)
            x2 = jnp.where(sidew, a1, b1)
            y2 = jnp.where(sidew, b1, a1)
            x1 = jnp.maximum(jnp.minimum(x1, clipx), zero16)
            y1 = jnp.maximum(jnp.minimum(y1, clipy), zero16)
            x2 = jnp.maximum(jnp.minimum(x2, clipx), zero16)
            y2 = jnp.maximum(jnp.minimum(y2, clipy), zero16)
            st_idx = g * 48 + lanes * 3 + k
            plsc.store_scatter(stage_sc, [st_idx], sc_v)
            plsc.store_scatter(stage_bx, [st_idx * 4], x1)
            plsc.store_scatter(stage_bx, [st_idx * 4 + 1], y1)
            plsc.store_scatter(stage_bx, [st_idx * 4 + 2], x2)
            plsc.store_scatter(stage_bx, [st_idx * 4 + 3], y2)
        return carry

    lax.fori_loop(0, 32, body, jnp.int32(0))
    pltpu.sync_copy(stage_bx, bx_out.at[img, side, pl.ds(part * 6144, 6144)])
    pltpu.sync_copy(stage_sc, sc_out.at[img, side, pl.ds(part * 1536, 1536)])


def _pairs_sc(prop_w, prop_h, sc_w, sc_h, ia_w, sa_w, ia_h, sa_h, im_info):
    B = prop_w.shape[0]
    props = jnp.stack([prop_w.reshape(B, -1), prop_h.reshape(B, -1)], axis=1)
    scs = jnp.stack([sc_w, sc_h], axis=1)
    pad = ((0, 0), (0, 2048 - _COM_TOPN))
    ias = jnp.stack([jnp.pad(ia_w, pad), jnp.pad(ia_h, pad)], axis=1)
    sas = jnp.stack([jnp.pad(sa_w, pad), jnp.pad(sa_h, pad)], axis=1)
    clipb = jnp.stack([
        jnp.broadcast_to(im_info[:, 1][:, None] - 1.0, (B, 16)),
        jnp.broadcast_to(im_info[:, 0][:, None] - 1.0, (B, 16)),
    ], axis=1)

    run = pl.kernel(
        _pairs_sc_kernel,
        out_type=(
            jax.ShapeDtypeStruct((B, 2, 24576), jnp.float32),
            jax.ShapeDtypeStruct((B, 2, 6144), jnp.float32),
        ),
        scratch_types=[
            pltpu.VMEM((14336,), jnp.float32),
            pltpu.VMEM((14336,), jnp.float32),
            pltpu.VMEM((7168,), jnp.float32),
            pltpu.VMEM((512,), jnp.int32),
            pltpu.VMEM((512,), jnp.float32),
            pltpu.VMEM((6144,), jnp.float32),
            pltpu.VMEM((1536,), jnp.float32),
            pltpu.VMEM((2, 16), jnp.float32),
        ],
        mesh=plsc.VectorSubcoreMesh(core_axis_name="c", subcore_axis_name="s"),
        compiler_params=pltpu.CompilerParams(needs_layout_passes=False),
    )
    bx, scp = run(props, scs, ias, sas, clipb)
    boxes = bx.reshape(B, 2, 6144, 4)[:, :, :6000].reshape(B, 12000, 4)
    scores = scp[:, :, :6000].reshape(B, 12000)
    return boxes, scores


def _clip_boxes(boxes, im_info):
    h = im_info[:, 0][:, None]
    w = im_info[:, 1][:, None]
    x1 = jnp.clip(boxes[..., 0], 0.0, w - 1.0)
    y1 = jnp.clip(boxes[..., 1], 0.0, h - 1.0)
    x2 = jnp.clip(boxes[..., 2], 0.0, w - 1.0)
    y2 = jnp.clip(boxes[..., 3], 0.0, h - 1.0)
    return jnp.stack([x1, y1, x2, y2], axis=-1)


def _iou(colv, rowv):
    x1c, y1c, x2c, y2c, ac = colv
    x1r, y1r, x2r, y2r, ar = rowv
    xx1 = jnp.maximum(x1c, x1r)
    yy1 = jnp.maximum(y1c, y1r)
    xx2 = jnp.minimum(x2c, x2r)
    yy2 = jnp.minimum(y2c, y2r)
    iw = jnp.maximum(0.0, xx2 - xx1 + 1.0)
    ih = jnp.maximum(0.0, yy2 - yy1 + 1.0)
    inter = iw * ih
    return inter / (ac + ar - inter)


def _nms_kernel(col_ref, row_ref, out_ref, keep_col_ref):
    C, NC = _C, _NC
    out_ref[...] = jnp.zeros((1, NC, 1, C), jnp.float32)

    def get_col(t):
        return tuple(col_ref[0, t, :, k:k + 1] for k in range(5))

    def get_row(t):
        return tuple(row_ref[0, t, k:k + 1, :] for k in range(5))

    iota_s = lax.broadcasted_iota(jnp.int32, (C, C), 0)
    iota_l = lax.broadcasted_iota(jnp.int32, (C, C), 1)
    ident = iota_s == iota_l

    def chunk_step(carry):
        c, cnt = carry
        colc = get_col(c)
        rowc = get_row(c)

        def pbody(jt, sup):
            colj = get_col(jt)
            keepj = keep_col_ref[jt]
            s = (_iou(colj, rowc) > _THRESH) & (keepj > 0.5)
            return jnp.maximum(sup, jnp.max(jnp.where(s, 1.0, 0.0), axis=0, keepdims=True))

        sup_row = lax.fori_loop(0, c, pbody, jnp.zeros((1, C), jnp.float32))
        sup_col = jnp.max(jnp.where(ident, jnp.broadcast_to(sup_row, (C, C)), 0.0),
                          axis=1, keepdims=True)

        gt = _iou(colc, rowc) > _THRESH
        ma = jnp.where(gt & (iota_s > iota_l), 1.0, 0.0)
        mb = jnp.where(gt & (iota_l > iota_s), 1.0, 0.0)

        gidx_row = c * C + lax.broadcasted_iota(jnp.int32, (1, C), 1)
        gidx_col = c * C + lax.broadcasted_iota(jnp.int32, (C, 1), 0)
        pre_row = (1.0 - sup_row) * jnp.where(gidx_row < _N, 1.0, 0.0)
        pre_col = (1.0 - sup_col) * jnp.where(gidx_col < _N, 1.0, 0.0)

        def fcond(fc):
            return fc[2]

        def fbody(fc):
            k_row, _, _ = fc
            a1 = jnp.max(ma * k_row, axis=1, keepdims=True)
            k_col = pre_col * (1.0 - a1)
            a2 = jnp.max(mb * k_col, axis=0, keepdims=True)
            k_row_new = pre_row * (1.0 - a2)
            changed = jnp.max(jnp.abs(k_row_new - k_row)) > 0.0
            return (k_row_new, k_col, changed)

        k_row, k_col, _ = lax.while_loop(
            fcond, fbody,
            (pre_row, jnp.zeros((C, 1), jnp.float32), jnp.bool_(True)))

        keep_col_ref[c] = k_col
        out_ref[0, c] = k_row
        return (c + 1, cnt + jnp.sum(k_row))

    def ccond(carry):
        c, cnt = carry
        return (c < NC) & (cnt < float(_POST_TOPN))

    lax.while_loop(ccond, chunk_step, (jnp.int32(0), jnp.float32(0.0)))


def _nms_pallas(boxes_sorted):
    B = boxes_sorted.shape[0]
    bp = jnp.pad(boxes_sorted, ((0, 0), (0, _NP - _N), (0, 0)), constant_values=2e9)
    area = (bp[..., 2] - bp[..., 0] + 1.0) * (bp[..., 3] - bp[..., 1] + 1.0)
    feats = jnp.concatenate(
        [bp, area[..., None], jnp.zeros((B, _NP, 3), jnp.float32)], axis=-1)
    col3 = feats.reshape(B, _NC, _C, 8)
    row3 = jnp.transpose(feats, (0, 2, 1)).reshape(B, 8, _NC, _C).transpose(0, 2, 1, 3)

    keep4 = pl.pallas_call(
        _nms_kernel,
        grid=(B,),
        in_specs=[
            pl.BlockSpec((1, _NC, _C, 8), lambda b: (b, 0, 0, 0)),
            pl.BlockSpec((1, _NC, 8, _C), lambda b: (b, 0, 0, 0)),
        ],
        out_specs=pl.BlockSpec((1, _NC, 1, _C), lambda b: (b, 0, 0, 0)),
        out_shape=jax.ShapeDtypeStruct((B, _NC, 1, _C), jnp.float32),
        scratch_shapes=[pltpu.VMEM((_NC, _C, 1), jnp.float32)],
        interpret=_INTERPRET,
    )(col3, row3)
    return keep4.reshape(B, _NP)[:, :_N]


def kernel(scores_w, scores_h, bbox_deltas_w, bbox_deltas_h, im_info):
    B = scores_w.shape[0]
    A = _ANCH.shape[0]
    H, W = scores_w.shape[2], scores_w.shape[3]
    K = H * W
    anch = jnp.asarray(_ANCH)

    sc_w = jnp.transpose(scores_w[:, A:], (0, 2, 3, 1)).reshape(B, -1)
    sc_h = jnp.transpose(scores_h[:, A:], (0, 2, 3, 1)).reshape(B, -1)
    d_w = jnp.transpose(bbox_deltas_w, (0, 2, 3, 1)).reshape(B, -1, 2)
    d_h = jnp.transpose(bbox_deltas_h, (0, 2, 3, 1)).reshape(B, -1, 2)

    sx, sy = jnp.meshgrid(jnp.arange(W, dtype=jnp.float32) * _FEAT_STRIDE,
                          jnp.arange(H, dtype=jnp.float32) * _FEAT_STRIDE)
    shifts_x = sx.ravel()
    shifts_y = sy.ravel()
    anch_w = jnp.broadcast_to(
        (anch[None, :, :] + shifts_x[:, None, None]).reshape(1, K * A, 2), (B, K * A, 2))
    anch_h = jnp.broadcast_to(
        (anch[None, :, :] + shifts_y[:, None, None]).reshape(1, K * A, 2), (B, K * A, 2))
    prop_w = _decode(anch_w, d_w)
    prop_h = _decode(anch_h, d_h)
    sa_w, ia_w = jax.lax.top_k(sc_w, _COM_TOPN)
    sa_h, ia_h = jax.lax.top_k(sc_h, _COM_TOPN)
    proposals, scores = _pairs_sc(prop_w, prop_h, sc_w, sc_h,
                                  ia_w, sa_w, ia_h, sa_h, im_info)

    top_s, top_i = jax.lax.top_k(scores, _N)
    boxes_sorted = jnp.take_along_axis(proposals, top_i[..., None], axis=1)

    keep = _nms_pallas(boxes_sorted)

    arange = jnp.arange(_N, dtype=jnp.int32)
    key = jnp.where(keep > 0.5, arange[None, :], _N)
    vals, _ = jax.lax.top_k(_N - key, _POST_TOPN)
    idx = _N - vals
    validm = idx < _N
    idxc = jnp.minimum(idx, _N - 1)
    sel = jnp.take_along_axis(boxes_sorted, idxc[..., None], axis=1) \
        * validm[..., None].astype(jnp.float32)
    batch_col = jnp.broadcast_to(
        jnp.arange(B, dtype=sel.dtype)[:, None, None], (B, _POST_TOPN, 1))
    return jnp.concatenate([batch_col, sel], axis=2)


# speedup vs baseline: 1.2723x; 1.2723x over previous
import functools

import jax
import jax.numpy as jnp
import numpy as np
from jax import lax
from jax.experimental import pallas as pl
from jax.experimental.pallas import tpu as pltpu
from jax.experimental.pallas import tpu_sc as plsc

_FEAT_STRIDE = 16
_WH = np.array([16., 32., 64., 128., 256., 512., 1024.], dtype=np.float32)
_ANCH = np.stack([-(_WH - 1.0) / 2.0, (_WH - 1.0) / 2.0], axis=1).astype(np.float32)
_PRE_TOPN = 6000
_POST_TOPN = 300
_THRESH = 0.7
_COM_TOPN = 2000
_COM_TOPK = 3

_N = 6000
_C = 512
_NC = 12
_NP = _C * _NC

_INTERPRET = False


def _decode(strings, deltas):
    widths = strings[..., 1] - strings[..., 0] + 1.0
    ctr = strings[..., 0] + 0.5 * widths
    d_ctr = deltas[..., 0]
    d_w = jnp.clip(deltas[..., 1], -10.0, 4.0)
    pred_ctr = d_ctr * widths + ctr
    pred_w = jnp.exp(d_w) * widths
    return jnp.stack([pred_ctr - 0.5 * (pred_w - 1.0),
                      pred_ctr + 0.5 * (pred_w - 1.0)], axis=-1)


def _pairs(prop_a, prop_b, sc_a, sc_b, A, K, primary_is_w):
    B = prop_a.shape[0]
    top_sa, top_ia = jax.lax.top_k(sc_a, _COM_TOPN)
    pos = top_ia // A
    a_sel = jnp.take_along_axis(prop_a, top_ia[..., None], axis=1)
    sc_b_r = sc_b.reshape(B, K, A)
    prop_b_r = prop_b.reshape(B, K, A, 2)
    top_sb, top_ib = jax.lax.top_k(sc_b_r, _COM_TOPK)
    b_strings = jnp.take_along_axis(prop_b_r, top_ib[..., None], axis=2)
    idx_s = jnp.broadcast_to(pos[:, :, None], (B, _COM_TOPN, _COM_TOPK))
    sb_at = jnp.take_along_axis(top_sb, idx_s, axis=1)
    idx_b = jnp.broadcast_to(pos[:, :, None, None], (B, _COM_TOPN, _COM_TOPK, 2))
    b_at = jnp.take_along_axis(b_strings, idx_b, axis=1)
    a_exp = jnp.broadcast_to(a_sel[:, :, None, :], (B, _COM_TOPN, _COM_TOPK, 2))
    if primary_is_w:
        boxes = jnp.stack([a_exp[..., 0], b_at[..., 0], a_exp[..., 1], b_at[..., 1]], axis=-1)
    else:
        boxes = jnp.stack([b_at[..., 0], a_exp[..., 0], b_at[..., 1], a_exp[..., 1]], axis=-1)
    scores = top_sa[:, :, None] * sb_at
    return boxes.reshape(B, _COM_TOPN * _COM_TOPK, 4), scores.reshape(B, _COM_TOPN * _COM_TOPK)


def _pairs_sc_kernel(props, scs, ias, sas, clipb, bx_out, sc_out,
                     a_buf, b_buf, scb_buf, ia_buf, sa_buf,
                     stage_bx, stage_sc, clip_buf):
    cid = lax.axis_index("c")
    sid = lax.axis_index("s")
    wid = sid * 2 + cid
    img = wid // 8
    side = (wid // 4) % 2
    part = wid % 4

    pltpu.sync_copy(props.at[img, side], a_buf)
    pltpu.sync_copy(props.at[img, 1 - side], b_buf)
    pltpu.sync_copy(scs.at[img, 1 - side], scb_buf)
    pltpu.sync_copy(ias.at[img, side, pl.ds(part * 512, 512)], ia_buf)
    pltpu.sync_copy(sas.at[img, side, pl.ds(part * 512, 512)], sa_buf)
    pltpu.sync_copy(clipb.at[img], clip_buf)

    clipx = clip_buf[0, :]
    clipy = clip_buf[1, :]
    sidew = lax.broadcast(side, (16,)) == 0
    lanes = lax.broadcasted_iota(jnp.int32, (16,), 0)
    zero16 = jnp.zeros((16,), jnp.float32)

    def body(g, carry):
        ia_v = ia_buf[pl.ds(g * 16, 16)]
        sa_v = sa_buf[pl.ds(g * 16, 16)]
        a0 = plsc.load_gather(a_buf, [ia_v * 2])
        a1 = plsc.load_gather(a_buf, [ia_v * 2 + 1])
        pos7 = (ia_v // 7) * 7
        vals = [plsc.load_gather(scb_buf, [pos7 + a]) for a in range(7)]

        picked = []
        for _ in range(3):
            m = jnp.full((16,), -1.0, jnp.float32)
            mi = jnp.zeros((16,), jnp.int32)
            for a in range(7):
                va = vals[a]
                for (_, pix) in picked:
                    va = jnp.where(pix == a, -1.0, va)
                cond = va > m
                m = jnp.where(cond, va, m)
                mi = jnp.where(cond, a, mi)
            picked.append((m, mi))

        for k, (mk, ik) in enumerate(picked):
            bidx = pos7 + ik
            b0 = plsc.load_gather(b_buf, [bidx * 2])
            b1 = plsc.load_gather(b_buf, [bidx * 2 + 1])
            sc_v = sa_v * mk
            x1 = jnp.where(sidew, a0, b0)
            y1 = jnp.where(sidew, b0, a0)
            x2 = jnp.where(sidew, a1, b1)
            y2 = jnp.where(sidew, b1, a1)
            x1 = jnp.maximum(jnp.minimum(x1, clipx), zero16)
            y1 = jnp.maximum(jnp.minimum(y1, clipy), zero16)
            x2 = jnp.maximum(jnp.minimum(x2, clipx), zero16)
            y2 = jnp.maximum(jnp.minimum(y2, clipy), zero16)
            st_idx = g * 48 + lanes * 3 + k
            plsc.store_scatter(stage_sc, [st_idx], sc_v)
            plsc.store_scatter(stage_bx, [st_idx * 4], x1)
            plsc.store_scatter(stage_bx, [st_idx * 4 + 1], y1)
            plsc.store_scatter(stage_bx, [st_idx * 4 + 2], x2)
            plsc.store_scatter(stage_bx, [st_idx * 4 + 3], y2)
        return carry

    lax.fori_loop(0, 32, body, jnp.int32(0))
    pltpu.sync_copy(stage_bx, bx_out.at[img, side, pl.ds(part * 6144, 6144)])
    pltpu.sync_copy(stage_sc, sc_out.at[img, side, pl.ds(part * 1536, 1536)])


def _pairs_sc(prop_w, prop_h, sc_w, sc_h, ia_w, sa_w, ia_h, sa_h, im_info):
    B = prop_w.shape[0]
    props = jnp.stack([prop_w.reshape(B, -1), prop_h.reshape(B, -1)], axis=1)
    scs = jnp.stack([sc_w, sc_h], axis=1)
    pad = ((0, 0), (0, 2048 - _COM_TOPN))
    ias = jnp.stack([jnp.pad(ia_w, pad), jnp.pad(ia_h, pad)], axis=1)
    sas = jnp.stack([jnp.pad(sa_w, pad), jnp.pad(sa_h, pad)], axis=1)
    clipb = jnp.stack([
        jnp.broadcast_to(im_info[:, 1][:, None] - 1.0, (B, 16)),
        jnp.broadcast_to(im_info[:, 0][:, None] - 1.0, (B, 16)),
    ], axis=1)

    run = pl.kernel(
        _pairs_sc_kernel,
        out_type=(
            jax.ShapeDtypeStruct((B, 2, 24576), jnp.float32),
            jax.ShapeDtypeStruct((B, 2, 6144), jnp.float32),
        ),
        scratch_types=[
            pltpu.VMEM((14336,), jnp.float32),
            pltpu.VMEM((14336,), jnp.float32),
            pltpu.VMEM((7168,), jnp.float32),
            pltpu.VMEM((512,), jnp.int32),
            pltpu.VMEM((512,), jnp.float32),
            pltpu.VMEM((6144,), jnp.float32),
            pltpu.VMEM((1536,), jnp.float32),
            pltpu.VMEM((2, 16), jnp.float32),
        ],
        mesh=plsc.VectorSubcoreMesh(core_axis_name="c", subcore_axis_name="s"),
        compiler_params=pltpu.CompilerParams(needs_layout_passes=False),
    )
    bx, scp = run(props, scs, ias, sas, clipb)
    scores = scp[:, :, :6000].reshape(B, 12000)
    return bx.reshape(B, 49152), scores


def _layout_sc_kernel(bx, ti, col_out, row_out, bx_buf, ti_buf, colst, rowst):
    cid = lax.axis_index("c")
    sid = lax.axis_index("s")
    wid = sid * 2 + cid
    img = wid // 8
    part = wid % 8

    pltpu.sync_copy(bx.at[img], bx_buf)
    pltpu.sync_copy(ti.at[img, pl.ds(part * 768, 768)], ti_buf)

    even = (part % 2) == 0
    c_full = jnp.where(even, (3 * part) // 2, (3 * part + 1) // 2)
    c_half = jnp.where(even, (3 * part) // 2 + 1, (3 * part - 1) // 2)
    off_h = jnp.where(even, 0, 256)
    lanes = lax.broadcasted_iota(jnp.int32, (16,), 0)
    n0 = part * 768

    def body(g, carry):
        l = g * 16 + lanes
        n_v = n0 + l
        valid = n_v < _N
        p = ti_buf[pl.ds(g * 16, 16)]
        f = p * 4 + jnp.where(p >= 6000, 576, 0)
        x1 = plsc.load_gather(bx_buf, [f])
        y1 = plsc.load_gather(bx_buf, [f + 1])
        x2 = plsc.load_gather(bx_buf, [f + 2])
        y2 = plsc.load_gather(bx_buf, [f + 3])
        big = jnp.full((16,), 2e9, jnp.float32)
        x1 = jnp.where(valid, x1, big)
        y1 = jnp.where(valid, y1, big)
        x2 = jnp.where(valid, x2, big)
        y2 = jnp.where(valid, y2, big)
        area = (x2 - x1 + 1.0) * (y2 - y1 + 1.0)
        in_full = jnp.where(even, l < 512, l >= 256)
        rbase = jnp.where(in_full,
                          jnp.where(even, l, l - 256),
                          jnp.where(even, l - 512, l))
        for k, v in enumerate((x1, y1, x2, y2, area)):
            plsc.store_scatter(colst, [l * 8 + k], v)
            ridx = jnp.where(in_full, k * 512 + rbase, 4096 + k * 256 + rbase)
            plsc.store_scatter(rowst, [ridx], v)
        return carry

    lax.fori_loop(0, 48, body, jnp.int32(0))

    pltpu.sync_copy(colst, col_out.at[img, pl.ds(part * 6144, 6144)])
    pltpu.sync_copy(rowst.at[pl.ds(0, 4096)], row_out.at[img, pl.ds(c_full * 4096, 4096)])
    for k in range(8):
        pltpu.sync_copy(rowst.at[pl.ds(4096 + k * 256, 256)],
                        row_out.at[img, pl.ds(c_half * 4096 + k * 512 + off_h, 256)])


def _layout_sc(bx_flat, top_i):
    B = bx_flat.shape[0]
    ti = jnp.pad(top_i, ((0, 0), (0, _NP - _N)))
    run = pl.kernel(
        _layout_sc_kernel,
        out_type=(
            jax.ShapeDtypeStruct((B, 49152), jnp.float32),
            jax.ShapeDtypeStruct((B, 49152), jnp.float32),
        ),
        scratch_types=[
            pltpu.VMEM((49152,), jnp.float32),
            pltpu.VMEM((768,), jnp.int32),
            pltpu.VMEM((6144,), jnp.float32),
            pltpu.VMEM((6144,), jnp.float32),
        ],
        mesh=plsc.VectorSubcoreMesh(core_axis_name="c", subcore_axis_name="s"),
        compiler_params=pltpu.CompilerParams(needs_layout_passes=False),
    )
    col_f, row_f = run(bx_flat, ti)
    return col_f.reshape(B, _NC, _C, 8), row_f.reshape(B, _NC, 8, _C)


def _clip_boxes(boxes, im_info):
    h = im_info[:, 0][:, None]
    w = im_info[:, 1][:, None]
    x1 = jnp.clip(boxes[..., 0], 0.0, w - 1.0)
    y1 = jnp.clip(boxes[..., 1], 0.0, h - 1.0)
    x2 = jnp.clip(boxes[..., 2], 0.0, w - 1.0)
    y2 = jnp.clip(boxes[..., 3], 0.0, h - 1.0)
    return jnp.stack([x1, y1, x2, y2], axis=-1)


def _iou(colv, rowv):
    x1c, y1c, x2c, y2c, ac = colv
    x1r, y1r, x2r, y2r, ar = rowv
    xx1 = jnp.maximum(x1c, x1r)
    yy1 = jnp.maximum(y1c, y1r)
    xx2 = jnp.minimum(x2c, x2r)
    yy2 = jnp.minimum(y2c, y2r)
    iw = jnp.maximum(0.0, xx2 - xx1 + 1.0)
    ih = jnp.maximum(0.0, yy2 - yy1 + 1.0)
    inter = iw * ih
    return inter / (ac + ar - inter)


def _nms_kernel(col_ref, row_ref, out_ref, keep_col_ref):
    C, NC = _C, _NC
    out_ref[...] = jnp.zeros((1, NC, 1, C), jnp.float32)

    def get_col(t):
        return tuple(col_ref[0, t, :, k:k + 1] for k in range(5))

    def get_row(t):
        return tuple(row_ref[0, t, k:k + 1, :] for k in range(5))

    iota_s = lax.broadcasted_iota(jnp.int32, (C, C), 0)
    iota_l = lax.broadcasted_iota(jnp.int32, (C, C), 1)
    ident = iota_s == iota_l

    def chunk_step(carry):
        c, cnt = carry
        colc = get_col(c)
        rowc = get_row(c)

        def pbody(jt, sup):
            colj = get_col(jt)
            keepj = keep_col_ref[jt]
            s = (_iou(colj, rowc) > _THRESH) & (keepj > 0.5)
            return jnp.maximum(sup, jnp.max(jnp.where(s, 1.0, 0.0), axis=0, keepdims=True))

        sup_row = lax.fori_loop(0, c, pbody, jnp.zeros((1, C), jnp.float32))
        sup_col = jnp.max(jnp.where(ident, jnp.broadcast_to(sup_row, (C, C)), 0.0),
                          axis=1, keepdims=True)

        gt = _iou(colc, rowc) > _THRESH
        ma = jnp.where(gt & (iota_s > iota_l), 1.0, 0.0)
        mb = jnp.where(gt & (iota_l > iota_s), 1.0, 0.0)

        gidx_row = c * C + lax.broadcasted_iota(jnp.int32, (1, C), 1)
        gidx_col = c * C + lax.broadcasted_iota(jnp.int32, (C, 1), 0)
        pre_row = (1.0 - sup_row) * jnp.where(gidx_row < _N, 1.0, 0.0)
        pre_col = (1.0 - sup_col) * jnp.where(gidx_col < _N, 1.0, 0.0)

        def fcond(fc):
            return fc[2]

        def fbody(fc):
            k_row, _, _ = fc
            a1 = jnp.max(ma * k_row, axis=1, keepdims=True)
            k_col = pre_col * (1.0 - a1)
            a2 = jnp.max(mb * k_col, axis=0, keepdims=True)
            k_row_new = pre_row * (1.0 - a2)
            changed = jnp.max(jnp.abs(k_row_new - k_row)) > 0.0
            return (k_row_new, k_col, changed)

        k_row, k_col, _ = lax.while_loop(
            fcond, fbody,
            (pre_row, jnp.zeros((C, 1), jnp.float32), jnp.bool_(True)))

        keep_col_ref[c] = k_col
        out_ref[0, c] = k_row
        return (c + 1, cnt + jnp.sum(k_row))

    def ccond(carry):
        c, cnt = carry
        return (c < NC) & (cnt < float(_POST_TOPN))

    lax.while_loop(ccond, chunk_step, (jnp.int32(0), jnp.float32(0.0)))


def _nms_pallas(col3, row3):
    B = col3.shape[0]
    keep4 = pl.pallas_call(
        _nms_kernel,
        grid=(B,),
        in_specs=[
            pl.BlockSpec((1, _NC, _C, 8), lambda b: (b, 0, 0, 0)),
            pl.BlockSpec((1, _NC, 8, _C), lambda b: (b, 0, 0, 0)),
        ],
        out_specs=pl.BlockSpec((1, _NC, 1, _C), lambda b: (b, 0, 0, 0)),
        out_shape=jax.ShapeDtypeStruct((B, _NC, 1, _C), jnp.float32),
        scratch_shapes=[pltpu.VMEM((_NC, _C, 1), jnp.float32)],
        interpret=_INTERPRET,
    )(col3, row3)
    return keep4.reshape(B, _NP)[:, :_N]


def kernel(scores_w, scores_h, bbox_deltas_w, bbox_deltas_h, im_info):
    B = scores_w.shape[0]
    A = _ANCH.shape[0]
    H, W = scores_w.shape[2], scores_w.shape[3]
    K = H * W
    anch = jnp.asarray(_ANCH)

    sc_w = jnp.transpose(scores_w[:, A:], (0, 2, 3, 1)).reshape(B, -1)
    sc_h = jnp.transpose(scores_h[:, A:], (0, 2, 3, 1)).reshape(B, -1)
    d_w = jnp.transpose(bbox_deltas_w, (0, 2, 3, 1)).reshape(B, -1, 2)
    d_h = jnp.transpose(bbox_deltas_h, (0, 2, 3, 1)).reshape(B, -1, 2)

    sx, sy = jnp.meshgrid(jnp.arange(W, dtype=jnp.float32) * _FEAT_STRIDE,
                          jnp.arange(H, dtype=jnp.float32) * _FEAT_STRIDE)
    shifts_x = sx.ravel()
    shifts_y = sy.ravel()
    anch_w = jnp.broadcast_to(
        (anch[None, :, :] + shifts_x[:, None, None]).reshape(1, K * A, 2), (B, K * A, 2))
    anch_h = jnp.broadcast_to(
        (anch[None, :, :] + shifts_y[:, None, None]).reshape(1, K * A, 2), (B, K * A, 2))
    prop_w = _decode(anch_w, d_w)
    prop_h = _decode(anch_h, d_h)
    sa_w, ia_w = jax.lax.top_k(sc_w, _COM_TOPN)
    sa_h, ia_h = jax.lax.top_k(sc_h, _COM_TOPN)
    bx_flat, scores = _pairs_sc(prop_w, prop_h, sc_w, sc_h,
                                ia_w, sa_w, ia_h, sa_h, im_info)

    top_s, top_i = jax.lax.top_k(scores, _N)
    col3, row3 = _layout_sc(bx_flat, top_i)

    keep = _nms_pallas(col3, row3)

    arange = jnp.arange(_N, dtype=jnp.int32)
    key = jnp.where(keep > 0.5, arange[None, :], _N)
    vals, _ = jax.lax.top_k(_N - key, _POST_TOPN)
    idx = _N - vals
    validm = idx < _N
    idxc = jnp.minimum(idx, _N - 1)
    sel = jnp.take_along_axis(col3.reshape(B, _NP, 8), idxc[..., None], axis=1)[..., :4] \
        * validm[..., None].astype(jnp.float32)
    batch_col = jnp.broadcast_to(
        jnp.arange(B, dtype=sel.dtype)[:, None, None], (B, _POST_TOPN, 1))
    return jnp.concatenate([batch_col, sel], axis=2)


# speedup vs baseline: 1.3637x; 1.0718x over previous
import functools

import jax
import jax.numpy as jnp
import numpy as np
from jax import lax
from jax.experimental import pallas as pl
from jax.experimental.pallas import tpu as pltpu
from jax.experimental.pallas import tpu_sc as plsc

_FEAT_STRIDE = 16
_WH = np.array([16., 32., 64., 128., 256., 512., 1024.], dtype=np.float32)
_ANCH = np.stack([-(_WH - 1.0) / 2.0, (_WH - 1.0) / 2.0], axis=1).astype(np.float32)
_PRE_TOPN = 6000
_POST_TOPN = 300
_THRESH = 0.7
_COM_TOPN = 2000
_COM_TOPK = 3

_N = 6000
_C = 512
_NC = 12
_NP = _C * _NC
_R = 384

_INTERPRET = False


def _decode(strings, deltas):
    widths = strings[..., 1] - strings[..., 0] + 1.0
    ctr = strings[..., 0] + 0.5 * widths
    d_ctr = deltas[..., 0]
    d_w = jnp.clip(deltas[..., 1], -10.0, 4.0)
    pred_ctr = d_ctr * widths + ctr
    pred_w = jnp.exp(d_w) * widths
    return jnp.stack([pred_ctr - 0.5 * (pred_w - 1.0),
                      pred_ctr + 0.5 * (pred_w - 1.0)], axis=-1)


def _pairs(prop_a, prop_b, sc_a, sc_b, A, K, primary_is_w):
    B = prop_a.shape[0]
    top_sa, top_ia = jax.lax.top_k(sc_a, _COM_TOPN)
    pos = top_ia // A
    a_sel = jnp.take_along_axis(prop_a, top_ia[..., None], axis=1)
    sc_b_r = sc_b.reshape(B, K, A)
    prop_b_r = prop_b.reshape(B, K, A, 2)
    top_sb, top_ib = jax.lax.top_k(sc_b_r, _COM_TOPK)
    b_strings = jnp.take_along_axis(prop_b_r, top_ib[..., None], axis=2)
    idx_s = jnp.broadcast_to(pos[:, :, None], (B, _COM_TOPN, _COM_TOPK))
    sb_at = jnp.take_along_axis(top_sb, idx_s, axis=1)
    idx_b = jnp.broadcast_to(pos[:, :, None, None], (B, _COM_TOPN, _COM_TOPK, 2))
    b_at = jnp.take_along_axis(b_strings, idx_b, axis=1)
    a_exp = jnp.broadcast_to(a_sel[:, :, None, :], (B, _COM_TOPN, _COM_TOPK, 2))
    if primary_is_w:
        boxes = jnp.stack([a_exp[..., 0], b_at[..., 0], a_exp[..., 1], b_at[..., 1]], axis=-1)
    else:
        boxes = jnp.stack([b_at[..., 0], a_exp[..., 0], b_at[..., 1], a_exp[..., 1]], axis=-1)
    scores = top_sa[:, :, None] * sb_at
    return boxes.reshape(B, _COM_TOPN * _COM_TOPK, 4), scores.reshape(B, _COM_TOPN * _COM_TOPK)


def _pairs_sc_kernel(props, scs, ias, sas, clipb, bx_out, sc_out,
                     a_buf, b_buf, scb_buf, ia_buf, sa_buf,
                     stage_bx, stage_sc, clip_buf):
    cid = lax.axis_index("c")
    sid = lax.axis_index("s")
    wid = sid * 2 + cid
    img = wid // 8
    side = (wid // 4) % 2
    part = wid % 4

    pltpu.sync_copy(props.at[img, side], a_buf)
    pltpu.sync_copy(props.at[img, 1 - side], b_buf)
    pltpu.sync_copy(scs.at[img, 1 - side], scb_buf)
    pltpu.sync_copy(ias.at[img, side, pl.ds(part * 512, 512)], ia_buf)
    pltpu.sync_copy(sas.at[img, side, pl.ds(part * 512, 512)], sa_buf)
    pltpu.sync_copy(clipb.at[img], clip_buf)

    clipx = clip_buf[0, :]
    clipy = clip_buf[1, :]
    sidew = lax.broadcast(side, (16,)) == 0
    lanes = lax.broadcasted_iota(jnp.int32, (16,), 0)
    zero16 = jnp.zeros((16,), jnp.float32)

    def body(g, carry):
        ia_v = ia_buf[pl.ds(g * 16, 16)]
        sa_v = sa_buf[pl.ds(g * 16, 16)]
        a0 = plsc.load_gather(a_buf, [ia_v * 2])
        a1 = plsc.load_gather(a_buf, [ia_v * 2 + 1])
        pos7 = (ia_v // 7) * 7
        vals = [plsc.load_gather(scb_buf, [pos7 + a]) for a in range(7)]

        picked = []
        for _ in range(3):
            m = jnp.full((16,), -1.0, jnp.float32)
            mi = jnp.zeros((16,), jnp.int32)
            for a in range(7):
                va = vals[a]
                for (_, pix) in picked:
                    va = jnp.where(pix == a, -1.0, va)
                cond = va > m
                m = jnp.where(cond, va, m)
                mi = jnp.where(cond, a, mi)
            picked.append((m, mi))

        for k, (mk, ik) in enumerate(picked):
            bidx = pos7 + ik
            b0 = plsc.load_gather(b_buf, [bidx * 2])
            b1 = plsc.load_gather(b_buf, [bidx * 2 + 1])
            sc_v = sa_v * mk
            x1 = jnp.where(sidew, a0, b0)
            y1 = jnp.where(sidew, b0, a0)
            x2 = jnp.where(sidew, a1, b1)
            y2 = jnp.where(sidew, b1, a1)
            x1 = jnp.maximum(jnp.minimum(x1, clipx), zero16)
            y1 = jnp.maximum(jnp.minimum(y1, clipy), zero16)
            x2 = jnp.maximum(jnp.minimum(x2, clipx), zero16)
            y2 = jnp.maximum(jnp.minimum(y2, clipy), zero16)
            st_idx = g * 48 + lanes * 3 + k
            plsc.store_scatter(stage_sc, [st_idx], sc_v)
            plsc.store_scatter(stage_bx, [st_idx * 4], x1)
            plsc.store_scatter(stage_bx, [st_idx * 4 + 1], y1)
            plsc.store_scatter(stage_bx, [st_idx * 4 + 2], x2)
            plsc.store_scatter(stage_bx, [st_idx * 4 + 3], y2)
        return carry

    lax.fori_loop(0, 32, body, jnp.int32(0))
    pltpu.sync_copy(stage_bx, bx_out.at[img, side, pl.ds(part * 6144, 6144)])
    pltpu.sync_copy(stage_sc, sc_out.at[img, side, pl.ds(part * 1536, 1536)])


def _pairs_sc(prop_w, prop_h, sc_w, sc_h, ia_w, sa_w, ia_h, sa_h, im_info):
    B = prop_w.shape[0]
    props = jnp.stack([prop_w.reshape(B, -1), prop_h.reshape(B, -1)], axis=1)
    scs = jnp.stack([sc_w, sc_h], axis=1)
    pad = ((0, 0), (0, 2048 - _COM_TOPN))
    ias = jnp.stack([jnp.pad(ia_w, pad), jnp.pad(ia_h, pad)], axis=1)
    sas = jnp.stack([jnp.pad(sa_w, pad), jnp.pad(sa_h, pad)], axis=1)
    clipb = jnp.stack([
        jnp.broadcast_to(im_info[:, 1][:, None] - 1.0, (B, 16)),
        jnp.broadcast_to(im_info[:, 0][:, None] - 1.0, (B, 16)),
    ], axis=1)

    run = pl.kernel(
        _pairs_sc_kernel,
        out_type=(
            jax.ShapeDtypeStruct((B, 2, 24576), jnp.float32),
            jax.ShapeDtypeStruct((B, 2, 6144), jnp.float32),
        ),
        scratch_types=[
            pltpu.VMEM((14336,), jnp.float32),
            pltpu.VMEM((14336,), jnp.float32),
            pltpu.VMEM((7168,), jnp.float32),
            pltpu.VMEM((512,), jnp.int32),
            pltpu.VMEM((512,), jnp.float32),
            pltpu.VMEM((6144,), jnp.float32),
            pltpu.VMEM((1536,), jnp.float32),
            pltpu.VMEM((2, 16), jnp.float32),
        ],
        mesh=plsc.VectorSubcoreMesh(core_axis_name="c", subcore_axis_name="s"),
        compiler_params=pltpu.CompilerParams(needs_layout_passes=False),
    )
    bx, scp = run(props, scs, ias, sas, clipb)
    scores = scp[:, :, :6000].reshape(B, 12000)
    return bx.reshape(B, 49152), scores


def _layout_sc_kernel(bx, ti, col_out, row_out, bx_buf, ti_buf, colst, rowst):
    cid = lax.axis_index("c")
    sid = lax.axis_index("s")
    wid = sid * 2 + cid
    img = wid // 8
    part = wid % 8

    pltpu.sync_copy(bx.at[img], bx_buf)
    pltpu.sync_copy(ti.at[img, pl.ds(part * 768, 768)], ti_buf)

    even = (part % 2) == 0
    c_full = jnp.where(even, (3 * part) // 2, (3 * part + 1) // 2)
    c_half = jnp.where(even, (3 * part) // 2 + 1, (3 * part - 1) // 2)
    off_h = jnp.where(even, 0, 256)
    lanes = lax.broadcasted_iota(jnp.int32, (16,), 0)
    n0 = part * 768

    def body(g, carry):
        l = g * 16 + lanes
        n_v = n0 + l
        valid = n_v < _N
        p = ti_buf[pl.ds(g * 16, 16)]
        f = p * 4 + jnp.where(p >= 6000, 576, 0)
        x1 = plsc.load_gather(bx_buf, [f])
        y1 = plsc.load_gather(bx_buf, [f + 1])
        x2 = plsc.load_gather(bx_buf, [f + 2])
        y2 = plsc.load_gather(bx_buf, [f + 3])
        big = jnp.full((16,), 2e9, jnp.float32)
        x1 = jnp.where(valid, x1, big)
        y1 = jnp.where(valid, y1, big)
        x2 = jnp.where(valid, x2, big)
        y2 = jnp.where(valid, y2, big)
        area = (x2 - x1 + 1.0) * (y2 - y1 + 1.0)
        in_full = jnp.where(even, l < 512, l >= 256)
        rbase = jnp.where(in_full,
                          jnp.where(even, l, l - 256),
                          jnp.where(even, l - 512, l))
        plsc.store_scatter(colst, [l * 8], jnp.ones((16,), jnp.float32))
        for k, v in enumerate((x1, y1, x2, y2, area)):
            plsc.store_scatter(colst, [l * 8 + 1 + k], v)
            ridx = jnp.where(in_full, k * 512 + rbase, 4096 + k * 256 + rbase)
            plsc.store_scatter(rowst, [ridx], v)
        return carry

    lax.fori_loop(0, 48, body, jnp.int32(0))

    pltpu.sync_copy(colst, col_out.at[img, pl.ds(part * 6144, 6144)])
    pltpu.sync_copy(rowst.at[pl.ds(0, 4096)], row_out.at[img, pl.ds(c_full * 4096, 4096)])
    for k in range(8):
        pltpu.sync_copy(rowst.at[pl.ds(4096 + k * 256, 256)],
                        row_out.at[img, pl.ds(c_half * 4096 + k * 512 + off_h, 256)])


def _layout_sc(bx_flat, top_i):
    B = bx_flat.shape[0]
    ti = jnp.pad(top_i, ((0, 0), (0, _NP - _N)))
    run = pl.kernel(
        _layout_sc_kernel,
        out_type=(
            jax.ShapeDtypeStruct((B, 49152), jnp.float32),
            jax.ShapeDtypeStruct((B, 49152), jnp.float32),
        ),
        scratch_types=[
            pltpu.VMEM((49152,), jnp.float32),
            pltpu.VMEM((768,), jnp.int32),
            pltpu.VMEM((6144,), jnp.float32),
            pltpu.VMEM((6144,), jnp.float32),
        ],
        mesh=plsc.VectorSubcoreMesh(core_axis_name="c", subcore_axis_name="s"),
        compiler_params=pltpu.CompilerParams(needs_layout_passes=False),
    )
    col_f, row_f = run(bx_flat, ti)
    return col_f.reshape(B, _NC, _C, 8), row_f.reshape(B, _NC, 8, _C)


def _clip_boxes(boxes, im_info):
    h = im_info[:, 0][:, None]
    w = im_info[:, 1][:, None]
    x1 = jnp.clip(boxes[..., 0], 0.0, w - 1.0)
    y1 = jnp.clip(boxes[..., 1], 0.0, h - 1.0)
    x2 = jnp.clip(boxes[..., 2], 0.0, w - 1.0)
    y2 = jnp.clip(boxes[..., 3], 0.0, h - 1.0)
    return jnp.stack([x1, y1, x2, y2], axis=-1)


def _iou(colv, rowv):
    x1c, y1c, x2c, y2c, ac = colv
    x1r, y1r, x2r, y2r, ar = rowv
    xx1 = jnp.maximum(x1c, x1r)
    yy1 = jnp.maximum(y1c, y1r)
    xx2 = jnp.minimum(x2c, x2r)
    yy2 = jnp.minimum(y2c, y2r)
    iw = jnp.maximum(0.0, xx2 - xx1 + 1.0)
    ih = jnp.maximum(0.0, yy2 - yy1 + 1.0)
    inter = iw * ih
    return inter / (ac + ar - inter)


def _nms_kernel(col_ref, row_ref, out_ref, keep_col_ref):
    C, NC, R = _C, _NC, _R
    out_ref[...] = jnp.zeros((1, R, 8), jnp.float32)

    def get_col(t):
        return tuple(col_ref[0, t, :, k + 1:k + 2] for k in range(5))

    def get_row(t):
        return tuple(row_ref[0, t, k:k + 1, :] for k in range(5))

    iota_s = lax.broadcasted_iota(jnp.int32, (C, C), 0)
    iota_l = lax.broadcasted_iota(jnp.int32, (C, C), 1)
    ident = iota_s == iota_l
    tri = jnp.where(iota_s <= iota_l, 1.0, 0.0)
    iota_rc = lax.broadcasted_iota(jnp.int32, (R, C), 0)

    def chunk_step(carry):
        c, cnt = carry
        colc = get_col(c)
        rowc = get_row(c)

        def pbody(jt, sup):
            colj = get_col(jt)
            keepj = keep_col_ref[jt]
            s = (_iou(colj, rowc) > _THRESH) & (keepj > 0.5)
            return jnp.maximum(sup, jnp.max(jnp.where(s, 1.0, 0.0), axis=0, keepdims=True))

        sup_row = lax.fori_loop(0, c, pbody, jnp.zeros((1, C), jnp.float32))
        sup_col = jnp.max(jnp.where(ident, jnp.broadcast_to(sup_row, (C, C)), 0.0),
                          axis=1, keepdims=True)

        gt = _iou(colc, rowc) > _THRESH
        ma = jnp.where(gt & (iota_s > iota_l), 1.0, 0.0)
        mb = jnp.where(gt & (iota_l > iota_s), 1.0, 0.0)

        gidx_row = c * C + lax.broadcasted_iota(jnp.int32, (1, C), 1)
        gidx_col = c * C + lax.broadcasted_iota(jnp.int32, (C, 1), 0)
        pre_row = (1.0 - sup_row) * jnp.where(gidx_row < _N, 1.0, 0.0)
        pre_col = (1.0 - sup_col) * jnp.where(gidx_col < _N, 1.0, 0.0)

        def fcond(fc):
            return fc[2]

        def fbody(fc):
            k_row, _, _ = fc
            a1 = jnp.max(ma * k_row, axis=1, keepdims=True)
            k_col = pre_col * (1.0 - a1)
            a2 = jnp.max(mb * k_col, axis=0, keepdims=True)
            k_row_new = pre_row * (1.0 - a2)
            changed = jnp.max(jnp.abs(k_row_new - k_row)) > 0.0
            return (k_row_new, k_col, changed)

        k_row, k_col, _ = lax.while_loop(
            fcond, fbody,
            (pre_row, jnp.zeros((C, 1), jnp.float32), jnp.bool_(True)))

        keep_col_ref[c] = k_col

        incl = lax.dot_general(k_row, tri, (((1,), (0,)), ((), ())),
                               precision=lax.Precision.HIGHEST)
        rank = cnt + incl - 1.0
        onehot = jnp.where(
            (iota_rc == rank.astype(jnp.int32)) & (k_row > 0.5), 1.0, 0.0)
        feats = col_ref[0, c]
        out_ref[0] += lax.dot_general(onehot, feats, (((1,), (0,)), ((), ())),
                                      precision=lax.Precision.HIGHEST)
        return (c + 1, cnt + jnp.sum(k_row))

    def ccond(carry):
        c, cnt = carry
        return (c < NC) & (cnt < float(_POST_TOPN))

    lax.while_loop(ccond, chunk_step, (jnp.int32(0), jnp.float32(0.0)))
    bval = lax.convert_element_type(pl.program_id(0), jnp.float32)
    out_ref[0, :, 0:1] = jnp.full((R, 1), 1.0, jnp.float32) * bval


def _nms_pallas(col3, row3):
    B = col3.shape[0]
    out = pl.pallas_call(
        _nms_kernel,
        grid=(B,),
        in_specs=[
            pl.BlockSpec((1, _NC, _C, 8), lambda b: (b, 0, 0, 0)),
            pl.BlockSpec((1, _NC, 8, _C), lambda b: (b, 0, 0, 0)),
        ],
        out_specs=pl.BlockSpec((1, _R, 8), lambda b: (b, 0, 0)),
        out_shape=jax.ShapeDtypeStruct((B, _R, 8), jnp.float32),
        scratch_shapes=[pltpu.VMEM((_NC, _C, 1), jnp.float32)],
        interpret=_INTERPRET,
    )(col3, row3)
    return out


def kernel(scores_w, scores_h, bbox_deltas_w, bbox_deltas_h, im_info):
    B = scores_w.shape[0]
    A = _ANCH.shape[0]
    H, W = scores_w.shape[2], scores_w.shape[3]
    K = H * W
    anch = jnp.asarray(_ANCH)

    sc_w = jnp.transpose(scores_w[:, A:], (0, 2, 3, 1)).reshape(B, -1)
    sc_h = jnp.transpose(scores_h[:, A:], (0, 2, 3, 1)).reshape(B, -1)
    d_w = jnp.transpose(bbox_deltas_w, (0, 2, 3, 1)).reshape(B, -1, 2)
    d_h = jnp.transpose(bbox_deltas_h, (0, 2, 3, 1)).reshape(B, -1, 2)

    sx, sy = jnp.meshgrid(jnp.arange(W, dtype=jnp.float32) * _FEAT_STRIDE,
                          jnp.arange(H, dtype=jnp.float32) * _FEAT_STRIDE)
    shifts_x = sx.ravel()
    shifts_y = sy.ravel()
    anch_w = jnp.broadcast_to(
        (anch[None, :, :] + shifts_x[:, None, None]).reshape(1, K * A, 2), (B, K * A, 2))
    anch_h = jnp.broadcast_to(
        (anch[None, :, :] + shifts_y[:, None, None]).reshape(1, K * A, 2), (B, K * A, 2))
    prop_w = _decode(anch_w, d_w)
    prop_h = _decode(anch_h, d_h)
    sa_w, ia_w = jax.lax.top_k(sc_w, _COM_TOPN)
    sa_h, ia_h = jax.lax.top_k(sc_h, _COM_TOPN)
    bx_flat, scores = _pairs_sc(prop_w, prop_h, sc_w, sc_h,
                                ia_w, sa_w, ia_h, sa_h, im_info)

    top_s, top_i = jax.lax.top_k(scores, _N)
    col3, row3 = _layout_sc(bx_flat, top_i)

    out = _nms_pallas(col3, row3)
    return out[:, :_POST_TOPN, :5]
